# baseline (speedup 1.0000x reference)
import functools
import jax
import jax.numpy as jnp
from jax import lax
from jax.experimental import pallas as pl
from jax.experimental.pallas import tpu as pltpu
from jax.experimental.pallas import tpu_sc as plsc

_N = 10000
_E = 320000
_D = 128
_DE = 16
_HID = 256
_NC = 10
_G = 64

_NPAD = 10240
_CROWS = _NPAD // 128
_NW = 32
_EPW = _E // _NW
_CH = 40
_ITERS = _EPW // _CH
_RPT = _NPAD // 16

_LOG1P_C = (-0.008466129016818367, 0.0436580512857575, -0.10679717589934831,
            0.1765967880850523, -0.24453302495503662, 0.3326523501519017,
            -0.49996354303547863, 0.9999995178202268)


def _tables_body(x_ref, wd_ref, ws_ref, td_ref, ts_ref):
    xb = x_ref[...]
    td_ref[...] = jnp.dot(xb, wd_ref[...], preferred_element_type=jnp.float32)
    ts_ref[...] = jnp.dot(xb, ws_ref[...], preferred_element_type=jnp.float32)


def _tc_tables(x, wd, ws):
    blk = 1000
    grid = _N // blk
    return pl.pallas_call(
        _tables_body,
        grid=(grid,),
        in_specs=[
            pl.BlockSpec((blk, _D), lambda i: (i, 0)),
            pl.BlockSpec((_D, 2 * _D), lambda i: (0, 0)),
            pl.BlockSpec((_D, 2 * _D), lambda i: (0, 0)),
        ],
        out_specs=[
            pl.BlockSpec((blk, 2 * _D), lambda i: (i, 0)),
            pl.BlockSpec((blk, 2 * _D), lambda i: (i, 0)),
        ],
        out_shape=[
            jax.ShapeDtypeStruct((_N, 2 * _D), jnp.float32),
            jax.ShapeDtypeStruct((_N, 2 * _D), jnp.float32),
        ],
    )(x, wd, ws)


def _ea_body(ea_ref, we1_ref, bb1_ref, we2_ref, bb2_ref, o1_ref, o2_ref):
    eb = ea_ref[...]
    o1_ref[...] = jnp.dot(eb, we1_ref[...], preferred_element_type=jnp.float32) + bb1_ref[...]
    o2_ref[...] = jnp.dot(eb, we2_ref[...], preferred_element_type=jnp.float32) + bb2_ref[...]


def _tc_edge_terms(edge_attr, we1, bb1, we2, bb2):
    blk = 4000
    grid = _E // blk
    return pl.pallas_call(
        _ea_body,
        grid=(grid,),
        in_specs=[
            pl.BlockSpec((blk, _DE), lambda i: (i, 0)),
            pl.BlockSpec((_DE, 2 * _D), lambda i: (0, 0)),
            pl.BlockSpec((1, 2 * _D), lambda i: (0, 0)),
            pl.BlockSpec((_DE, 2 * _D), lambda i: (0, 0)),
            pl.BlockSpec((1, 2 * _D), lambda i: (0, 0)),
        ],
        out_specs=[
            pl.BlockSpec((blk, 2 * _D), lambda i: (i, 0)),
            pl.BlockSpec((blk, 2 * _D), lambda i: (i, 0)),
        ],
        out_shape=[
            jax.ShapeDtypeStruct((_E, 2 * _D), jnp.float32),
            jax.ShapeDtypeStruct((_E, 2 * _D), jnp.float32),
        ],
    )(edge_attr, we1, bb1, we2, bb2)


def _softplus16(b):
    t = jnp.exp(-jnp.abs(b))
    p = jnp.full((16,), _LOG1P_C[0], dtype=jnp.float32)
    for c in _LOG1P_C[1:]:
        p = p * t + c
    return jnp.maximum(b, 0.0) + t * p


def _sc_edge_body(td_hbm, ts_hbm, ea_hbm, dst_hbm, src_hbm, zero_hbm,
                  out_hbm, dstv, srcv, gd, gs, ea, mbuf, acc):
    c = lax.axis_index("c")
    s = lax.axis_index("s")
    wid = s * 2 + c

    pltpu.sync_copy(zero_hbm.at[pl.ds(s * _RPT, _RPT)], acc.at[pl.ds(s * _RPT, _RPT)])
    plsc.subcore_barrier()

    def chunk(i, _):
        off = wid * _EPW + i * _CH
        pltpu.sync_copy(dst_hbm.at[pl.ds(off, _CH)], dstv)
        pltpu.sync_copy(src_hbm.at[pl.ds(off, _CH)], srcv)
        pltpu.sync_copy(td_hbm.at[dstv], gd)
        pltpu.sync_copy(ts_hbm.at[srcv], gs)
        pltpu.sync_copy(ea_hbm.at[pl.ds(off, _CH)], ea)

        def edge(e, _):
            for g in range(8):
                lo = pl.ds(g * 16, 16)
                hi = pl.ds(128 + g * 16, 16)
                a = gd[e, lo] + gs[e, lo] + ea[e, lo]
                b = gd[e, hi] + gs[e, hi] + ea[e, hi]
                f = 1.0 / (1.0 + jnp.exp(-a))
                mbuf[e, lo] = f * _softplus16(b)
            return _

        lax.fori_loop(0, _CH, edge, None)
        pltpu.sync_copy(mbuf, acc.at[dstv], add=True)
        return _

    lax.fori_loop(0, _ITERS, chunk, None)
    plsc.subcore_barrier()
    pltpu.sync_copy(acc.at[pl.ds(s * _RPT, _RPT)],
                    out_hbm.at[c, pl.ds(s * _RPT, _RPT)])


def _sc_edge_pass(td, ts, eat, dst, src, zrows):
    mesh = plsc.VectorSubcoreMesh(core_axis_name="c", subcore_axis_name="s")
    kfn = pl.kernel(
        _sc_edge_body,
        mesh=mesh,
        out_type=jax.ShapeDtypeStruct((2, _NPAD, _D), jnp.float32),
        scratch_types=[
            pltpu.VMEM((_CH,), jnp.int32),
            pltpu.VMEM((_CH,), jnp.int32),
            pltpu.VMEM((_CH, 2 * _D), jnp.float32),
            pltpu.VMEM((_CH, 2 * _D), jnp.float32),
            pltpu.VMEM((_CH, 2 * _D), jnp.float32),
            pltpu.VMEM((_CH, _D), jnp.float32),
            pltpu.VMEM_SHARED((_NPAD, _D), jnp.float32),
        ],
    )
    return kfn(td, ts, eat, dst, src, zrows)


def _sc_count_body(dst_hbm, zero_hbm, cout_hbm, dstv, onesb, cacc):
    c = lax.axis_index("c")
    s = lax.axis_index("s")
    wid = s * 2 + c

    pltpu.sync_copy(zero_hbm.at[pl.ds(s * _RPT, _RPT)], cacc.at[pl.ds(s * _RPT, _RPT)])
    ones16 = jnp.full((16,), 1.0, dtype=jnp.float32)

    def fill(r, _):
        for g in range(8):
            onesb[r, pl.ds(g * 16, 16)] = ones16
        return _

    lax.fori_loop(0, _CH, fill, None)
    plsc.subcore_barrier()

    def chunk(i, _):
        off = wid * _EPW + i * _CH
        pltpu.sync_copy(dst_hbm.at[pl.ds(off, _CH)], dstv)
        pltpu.sync_copy(onesb, cacc.at[dstv], add=True)
        return _

    lax.fori_loop(0, _ITERS, chunk, None)
    plsc.subcore_barrier()
    pltpu.sync_copy(cacc.at[pl.ds(s * _RPT, _RPT)],
                    cout_hbm.at[c, pl.ds(s * _RPT, _RPT)])


def _sc_count_pass(dst, zrows):
    mesh = plsc.VectorSubcoreMesh(core_axis_name="c", subcore_axis_name="s")
    kfn = pl.kernel(
        _sc_count_body,
        mesh=mesh,
        out_type=jax.ShapeDtypeStruct((2, _NPAD, _D), jnp.float32),
        scratch_types=[
            pltpu.VMEM((_CH,), jnp.int32),
            pltpu.VMEM((_CH, _D), jnp.float32),
            pltpu.VMEM_SHARED((_NPAD, _D), jnp.float32),
        ],
    )
    return kfn(dst, zrows)


def _combine_body(x_ref, p_ref, c_ref, wd_ref, ws_ref, h_ref, td_ref, ts_ref):
    p = p_ref[0] + p_ref[1]
    cnt = (c_ref[0] + c_ref[1])[:, 0:1]
    agg = p / jnp.maximum(cnt, 1.0)
    h = x_ref[...] + agg
    h_ref[...] = h
    td_ref[...] = jnp.dot(h, wd_ref[...], preferred_element_type=jnp.float32)
    ts_ref[...] = jnp.dot(h, ws_ref[...], preferred_element_type=jnp.float32)


def _tc_combine(x, parts, cnt4, wd2, ws2):
    blk = 1000
    grid = _N // blk
    return pl.pallas_call(
        _combine_body,
        grid=(grid,),
        in_specs=[
            pl.BlockSpec((blk, _D), lambda i: (i, 0)),
            pl.BlockSpec((2, blk, _D), lambda i: (0, i, 0)),
            pl.BlockSpec((2, blk, _D), lambda i: (0, i, 0)),
            pl.BlockSpec((_D, 2 * _D), lambda i: (0, 0)),
            pl.BlockSpec((_D, 2 * _D), lambda i: (0, 0)),
        ],
        out_specs=[
            pl.BlockSpec((blk, _D), lambda i: (i, 0)),
            pl.BlockSpec((blk, 2 * _D), lambda i: (i, 0)),
            pl.BlockSpec((blk, 2 * _D), lambda i: (i, 0)),
        ],
        out_shape=[
            jax.ShapeDtypeStruct((_N, _D), jnp.float32),
            jax.ShapeDtypeStruct((_N, 2 * _D), jnp.float32),
            jax.ShapeDtypeStruct((_N, 2 * _D), jnp.float32),
        ],
    )(x, parts, cnt4, wd2, ws2)


def _final_body(h1_ref, q_ref, cq_ref, b_ref, w1_ref, b1_ref, w2_ref, b2_ref,
                o_ref, s_acc, c_acc):
    i = pl.program_id(0)
    q = q_ref[0] + q_ref[1]
    cnt = (cq_ref[0] + cq_ref[1])[:, 0:1]
    h2 = h1_ref[...] + q / jnp.maximum(cnt, 1.0)

    batch = b_ref[0, 0, :].reshape(-1, 1)
    gids = lax.broadcasted_iota(jnp.int32, (batch.shape[0], _G), 1)
    oh = (batch == gids).astype(jnp.float32)
    dnum = (((0,), (0,)), ((), ()))
    sblk = lax.dot_general(oh, h2, dnum, preferred_element_type=jnp.float32)
    cblk = lax.dot_general(oh, jnp.ones_like(h2), dnum,
                           preferred_element_type=jnp.float32)

    @pl.when(i == 0)
    def _():
        s_acc[...] = jnp.zeros_like(s_acc)
        c_acc[...] = jnp.zeros_like(c_acc)

    s_acc[...] += sblk
    c_acc[...] += cblk

    @pl.when(i == pl.num_programs(0) - 1)
    def _():
        g = s_acc[...] / jnp.maximum(c_acc[...], 1.0)
        a1 = jnp.maximum(
            jnp.dot(g, w1_ref[...], preferred_element_type=jnp.float32)
            + b1_ref[...], 0.0)
        o = (jnp.dot(a1, w2_ref[...], preferred_element_type=jnp.float32)
             + b2_ref[...])
        mx = jnp.max(o, axis=1, keepdims=True)
        lse = jnp.log(jnp.sum(jnp.exp(o - mx), axis=1, keepdims=True)) + mx
        o_ref[...] = o - lse


def _tc_final(h1, parts2, cnt4, batch3, w1, b1, w2, b2):
    blk = 1000
    grid = _N // blk
    return pl.pallas_call(
        _final_body,
        grid=(grid,),
        in_specs=[
            pl.BlockSpec((blk, _D), lambda i: (i, 0)),
            pl.BlockSpec((2, blk, _D), lambda i: (0, i, 0)),
            pl.BlockSpec((2, blk, _D), lambda i: (0, i, 0)),
            pl.BlockSpec((1, 1, blk), lambda i: (i, 0, 0)),
            pl.BlockSpec((_D, _HID), lambda i: (0, 0)),
            pl.BlockSpec((1, _HID), lambda i: (0, 0)),
            pl.BlockSpec((_HID, _NC), lambda i: (0, 0)),
            pl.BlockSpec((1, _NC), lambda i: (0, 0)),
        ],
        out_specs=pl.BlockSpec((_G, _NC), lambda i: (0, 0)),
        out_shape=jax.ShapeDtypeStruct((_G, _NC), jnp.float32),
        scratch_shapes=[
            pltpu.VMEM((_G, _D), jnp.float32),
            pltpu.VMEM((_G, _D), jnp.float32),
        ],
    )(h1, parts2, cnt4, batch3, w1, b1, w2, b2)


@jax.jit
def kernel(x, edge_index, edge_attr, batch, Wf1, bf1, Ws1, bs1,
           Wf2, bf2, Ws2, bs2, W1, b1, W2, b2):
    wd1 = jnp.concatenate([Wf1[:_D], Ws1[:_D]], axis=1)
    wsr1 = jnp.concatenate([Wf1[_D:2 * _D], Ws1[_D:2 * _D]], axis=1)
    we1 = jnp.concatenate([Wf1[2 * _D:], Ws1[2 * _D:]], axis=1)
    bb1 = jnp.concatenate([bf1, bs1]).reshape(1, 2 * _D)
    wd2 = jnp.concatenate([Wf2[:_D], Ws2[:_D]], axis=1)
    wsr2 = jnp.concatenate([Wf2[_D:2 * _D], Ws2[_D:2 * _D]], axis=1)
    we2 = jnp.concatenate([Wf2[2 * _D:], Ws2[2 * _D:]], axis=1)
    bb2 = jnp.concatenate([bf2, bs2]).reshape(1, 2 * _D)

    src = edge_index[0]
    dst = edge_index[1]
    zrows = jnp.zeros((_NPAD, _D), jnp.float32)
    batch3 = batch.reshape(_N // 1000, 1, 1000)

    td1, ts1 = _tc_tables(x, wd1, wsr1)
    ea1, ea2 = _tc_edge_terms(edge_attr, we1, bb1, we2, bb2)
    cparts = _sc_count_pass(dst, zrows)
    parts1 = _sc_edge_pass(td1, ts1, ea1, dst, src, zrows)
    h1, td2, ts2 = _tc_combine(x, parts1, cparts, wd2, wsr2)
    parts2 = _sc_edge_pass(td2, ts2, ea2, dst, src, zrows)
    return _tc_final(h1, parts2, cparts, batch3, W1, b1.reshape(1, _HID),
                     W2, b2.reshape(1, _NC))


# speedup vs baseline: 1.2491x; 1.2491x over previous
import functools
import jax
import jax.numpy as jnp
from jax import lax
from jax.experimental import pallas as pl
from jax.experimental.pallas import tpu as pltpu
from jax.experimental.pallas import tpu_sc as plsc

_N = 10000
_E = 320000
_D = 128
_DE = 16
_HID = 256
_NC = 10
_G = 64

_NPAD = 10240
_CROWS = _NPAD // 128
_NW = 32
_EPW = _E // _NW
_CH = 16
_ITERS = _EPW // _CH
_RPT = _NPAD // 16

_LOG1P_C = (-0.008466129016818367, 0.0436580512857575, -0.10679717589934831,
            0.1765967880850523, -0.24453302495503662, 0.3326523501519017,
            -0.49996354303547863, 0.9999995178202268)


def _tables_body(x_ref, wd_ref, ws_ref, td_ref, ts_ref):
    xb = x_ref[...]
    td_ref[...] = jnp.dot(xb, wd_ref[...], preferred_element_type=jnp.float32)
    ts_ref[...] = jnp.dot(xb, ws_ref[...], preferred_element_type=jnp.float32)


def _tc_tables(x, wd, ws):
    blk = 1000
    grid = _N // blk
    return pl.pallas_call(
        _tables_body,
        grid=(grid,),
        in_specs=[
            pl.BlockSpec((blk, _D), lambda i: (i, 0)),
            pl.BlockSpec((_D, 2 * _D), lambda i: (0, 0)),
            pl.BlockSpec((_D, 2 * _D), lambda i: (0, 0)),
        ],
        out_specs=[
            pl.BlockSpec((blk, 2 * _D), lambda i: (i, 0)),
            pl.BlockSpec((blk, 2 * _D), lambda i: (i, 0)),
        ],
        out_shape=[
            jax.ShapeDtypeStruct((_N, 2 * _D), jnp.float32),
            jax.ShapeDtypeStruct((_N, 2 * _D), jnp.float32),
        ],
    )(x, wd, ws)


def _ea_body(ea_ref, we1_ref, bb1_ref, we2_ref, bb2_ref, o1_ref, o2_ref):
    eb = ea_ref[...]
    o1_ref[...] = jnp.dot(eb, we1_ref[...], preferred_element_type=jnp.float32) + bb1_ref[...]
    o2_ref[...] = jnp.dot(eb, we2_ref[...], preferred_element_type=jnp.float32) + bb2_ref[...]


def _tc_edge_terms(edge_attr, we1, bb1, we2, bb2):
    blk = 4000
    grid = _E // blk
    return pl.pallas_call(
        _ea_body,
        grid=(grid,),
        in_specs=[
            pl.BlockSpec((blk, _DE), lambda i: (i, 0)),
            pl.BlockSpec((_DE, 2 * _D), lambda i: (0, 0)),
            pl.BlockSpec((1, 2 * _D), lambda i: (0, 0)),
            pl.BlockSpec((_DE, 2 * _D), lambda i: (0, 0)),
            pl.BlockSpec((1, 2 * _D), lambda i: (0, 0)),
        ],
        out_specs=[
            pl.BlockSpec((blk, 2 * _D), lambda i: (i, 0)),
            pl.BlockSpec((blk, 2 * _D), lambda i: (i, 0)),
        ],
        out_shape=[
            jax.ShapeDtypeStruct((_E, 2 * _D), jnp.float32),
            jax.ShapeDtypeStruct((_E, 2 * _D), jnp.float32),
        ],
    )(edge_attr, we1, bb1, we2, bb2)


def _softplus16(b):
    t = jnp.exp(-jnp.abs(b))
    p = jnp.full((16,), _LOG1P_C[0], dtype=jnp.float32)
    for c in _LOG1P_C[1:]:
        p = p * t + c
    return jnp.maximum(b, 0.0) + t * p


def _sc_edge_body(td_hbm, ts_hbm, ea_hbm, dst_hbm, src_hbm, zero_hbm,
                  out_hbm, dstv, srcv, dsts, gd, gs, ea, mbuf, acc,
                  semi, semg, sems):
    c = lax.axis_index("c")
    s = lax.axis_index("s")
    wid = s * 2 + c
    base = wid * _EPW
    last = _ITERS - 1

    pltpu.sync_copy(zero_hbm.at[pl.ds(s * _RPT, _RPT)], acc.at[pl.ds(s * _RPT, _RPT)])
    plsc.subcore_barrier()

    def idx_start(i, p):
        off = base + i * _CH
        pltpu.async_copy(dst_hbm.at[pl.ds(off, _CH)], dstv.at[p], semi.at[p])
        pltpu.async_copy(src_hbm.at[pl.ds(off, _CH)], srcv.at[p], semi.at[p])

    def idx_wait(p):
        pltpu.make_async_copy(dst_hbm.at[pl.ds(0, _CH)], dstv.at[p], semi.at[p]).wait()
        pltpu.make_async_copy(src_hbm.at[pl.ds(0, _CH)], srcv.at[p], semi.at[p]).wait()

    def gather_start(i, p):
        off = base + i * _CH
        pltpu.async_copy(td_hbm.at[dstv.at[p]], gd.at[p], semg.at[p])
        pltpu.async_copy(ts_hbm.at[srcv.at[p]], gs.at[p], semg.at[p])
        pltpu.async_copy(ea_hbm.at[pl.ds(off, _CH)], ea.at[p], semg.at[p])

    def gather_wait(p):
        pltpu.make_async_copy(td_hbm.at[dstv.at[p]], gd.at[p], semg.at[p]).wait()
        pltpu.make_async_copy(ts_hbm.at[srcv.at[p]], gs.at[p], semg.at[p]).wait()
        pltpu.make_async_copy(ea_hbm.at[pl.ds(0, _CH)], ea.at[p], semg.at[p]).wait()

    def scatter_start(p):
        pltpu.async_copy(mbuf.at[p], acc.at[dsts.at[p]], sems.at[p], add=True)

    def scatter_wait(p):
        pltpu.make_async_copy(mbuf.at[p], acc.at[dsts.at[p]], sems.at[p]).wait()

    def compute(p):
        def edge(e, _):
            for g in range(8):
                lo = pl.ds(g * 16, 16)
                hi = pl.ds(128 + g * 16, 16)
                a = gd[p, e, lo] + gs[p, e, lo] + ea[p, e, lo]
                b = gd[p, e, hi] + gs[p, e, hi] + ea[p, e, hi]
                f = 1.0 / (1.0 + jnp.exp(-a))
                mbuf[p, e, lo] = f * _softplus16(b)
            return _

        lax.fori_loop(0, _CH, edge, None)

    idx_start(0, 0)
    idx_wait(0)
    gather_start(0, 0)
    idx_start(1, 1)

    def pair(k, _):
        for p in (0, 1):
            i = 2 * k + p

            @pl.when((i >= 2) & (i <= last))
            def _():
                scatter_wait(p)

            @pl.when(i <= last)
            def _():
                gather_wait(p)
                dsts[p, pl.ds(0, _CH)] = dstv[p, pl.ds(0, _CH)]

            @pl.when(i + 2 <= last)
            def _():
                idx_start(i + 2, p)

            @pl.when(i + 1 <= last)
            def _():
                idx_wait(1 - p)
                gather_start(i + 1, 1 - p)

            @pl.when(i <= last)
            def _():
                compute(p)
                scatter_start(p)
        return _

    lax.fori_loop(0, (_ITERS + 1) // 2, pair, None)
    scatter_wait(last % 2)
    scatter_wait(1 - last % 2)
    plsc.subcore_barrier()
    pltpu.sync_copy(acc.at[pl.ds(s * _RPT, _RPT)],
                    out_hbm.at[c, pl.ds(s * _RPT, _RPT)])


def _sc_edge_pass(td, ts, eat, dst, src, zrows):
    mesh = plsc.VectorSubcoreMesh(core_axis_name="c", subcore_axis_name="s")
    kfn = pl.kernel(
        _sc_edge_body,
        mesh=mesh,
        out_type=jax.ShapeDtypeStruct((2, _NPAD, _D), jnp.float32),
        scratch_types=[
            pltpu.VMEM((2, _CH), jnp.int32),
            pltpu.VMEM((2, _CH), jnp.int32),
            pltpu.VMEM((2, _CH), jnp.int32),
            pltpu.VMEM((2, _CH, 2 * _D), jnp.float32),
            pltpu.VMEM((2, _CH, 2 * _D), jnp.float32),
            pltpu.VMEM((2, _CH, 2 * _D), jnp.float32),
            pltpu.VMEM((2, _CH, _D), jnp.float32),
            pltpu.VMEM_SHARED((_NPAD, _D), jnp.float32),
            pltpu.SemaphoreType.DMA((2,)),
            pltpu.SemaphoreType.DMA((2,)),
            pltpu.SemaphoreType.DMA((2,)),
        ],
    )
    return kfn(td, ts, eat, dst, src, zrows)


def _sc_count_body(dst_hbm, zero_hbm, cout_hbm, dstv, onesb, cacc):
    c = lax.axis_index("c")
    s = lax.axis_index("s")
    wid = s * 2 + c

    pltpu.sync_copy(zero_hbm.at[pl.ds(s * _RPT, _RPT)], cacc.at[pl.ds(s * _RPT, _RPT)])
    ones16 = jnp.full((16,), 1.0, dtype=jnp.float32)

    def fill(r, _):
        for g in range(8):
            onesb[r, pl.ds(g * 16, 16)] = ones16
        return _

    lax.fori_loop(0, _CH, fill, None)
    plsc.subcore_barrier()

    def chunk(i, _):
        off = wid * _EPW + i * _CH
        pltpu.sync_copy(dst_hbm.at[pl.ds(off, _CH)], dstv)
        pltpu.sync_copy(onesb, cacc.at[dstv], add=True)
        return _

    lax.fori_loop(0, _ITERS, chunk, None)
    plsc.subcore_barrier()
    pltpu.sync_copy(cacc.at[pl.ds(s * _RPT, _RPT)],
                    cout_hbm.at[c, pl.ds(s * _RPT, _RPT)])


def _sc_count_pass(dst, zrows):
    mesh = plsc.VectorSubcoreMesh(core_axis_name="c", subcore_axis_name="s")
    kfn = pl.kernel(
        _sc_count_body,
        mesh=mesh,
        out_type=jax.ShapeDtypeStruct((2, _NPAD, _D), jnp.float32),
        scratch_types=[
            pltpu.VMEM((_CH,), jnp.int32),
            pltpu.VMEM((_CH, _D), jnp.float32),
            pltpu.VMEM_SHARED((_NPAD, _D), jnp.float32),
        ],
    )
    return kfn(dst, zrows)


def _combine_body(x_ref, p_ref, c_ref, wd_ref, ws_ref, h_ref, td_ref, ts_ref):
    p = p_ref[0] + p_ref[1]
    cnt = (c_ref[0] + c_ref[1])[:, 0:1]
    agg = p / jnp.maximum(cnt, 1.0)
    h = x_ref[...] + agg
    h_ref[...] = h
    td_ref[...] = jnp.dot(h, wd_ref[...], preferred_element_type=jnp.float32)
    ts_ref[...] = jnp.dot(h, ws_ref[...], preferred_element_type=jnp.float32)


def _tc_combine(x, parts, cnt4, wd2, ws2):
    blk = 1000
    grid = _N // blk
    return pl.pallas_call(
        _combine_body,
        grid=(grid,),
        in_specs=[
            pl.BlockSpec((blk, _D), lambda i: (i, 0)),
            pl.BlockSpec((2, blk, _D), lambda i: (0, i, 0)),
            pl.BlockSpec((2, blk, _D), lambda i: (0, i, 0)),
            pl.BlockSpec((_D, 2 * _D), lambda i: (0, 0)),
            pl.BlockSpec((_D, 2 * _D), lambda i: (0, 0)),
        ],
        out_specs=[
            pl.BlockSpec((blk, _D), lambda i: (i, 0)),
            pl.BlockSpec((blk, 2 * _D), lambda i: (i, 0)),
            pl.BlockSpec((blk, 2 * _D), lambda i: (i, 0)),
        ],
        out_shape=[
            jax.ShapeDtypeStruct((_N, _D), jnp.float32),
            jax.ShapeDtypeStruct((_N, 2 * _D), jnp.float32),
            jax.ShapeDtypeStruct((_N, 2 * _D), jnp.float32),
        ],
    )(x, parts, cnt4, wd2, ws2)


def _final_body(h1_ref, q_ref, cq_ref, b_ref, w1_ref, b1_ref, w2_ref, b2_ref,
                o_ref, s_acc, c_acc):
    i = pl.program_id(0)
    q = q_ref[0] + q_ref[1]
    cnt = (cq_ref[0] + cq_ref[1])[:, 0:1]
    h2 = h1_ref[...] + q / jnp.maximum(cnt, 1.0)

    batch = b_ref[0, 0, :].reshape(-1, 1)
    gids = lax.broadcasted_iota(jnp.int32, (batch.shape[0], _G), 1)
    oh = (batch == gids).astype(jnp.float32)
    dnum = (((0,), (0,)), ((), ()))
    sblk = lax.dot_general(oh, h2, dnum, preferred_element_type=jnp.float32)
    cblk = lax.dot_general(oh, jnp.ones_like(h2), dnum,
                           preferred_element_type=jnp.float32)

    @pl.when(i == 0)
    def _():
        s_acc[...] = jnp.zeros_like(s_acc)
        c_acc[...] = jnp.zeros_like(c_acc)

    s_acc[...] += sblk
    c_acc[...] += cblk

    @pl.when(i == pl.num_programs(0) - 1)
    def _():
        g = s_acc[...] / jnp.maximum(c_acc[...], 1.0)
        a1 = jnp.maximum(
            jnp.dot(g, w1_ref[...], preferred_element_type=jnp.float32)
            + b1_ref[...], 0.0)
        o = (jnp.dot(a1, w2_ref[...], preferred_element_type=jnp.float32)
             + b2_ref[...])
        mx = jnp.max(o, axis=1, keepdims=True)
        lse = jnp.log(jnp.sum(jnp.exp(o - mx), axis=1, keepdims=True)) + mx
        o_ref[...] = o - lse


def _tc_final(h1, parts2, cnt4, batch3, w1, b1, w2, b2):
    blk = 1000
    grid = _N // blk
    return pl.pallas_call(
        _final_body,
        grid=(grid,),
        in_specs=[
            pl.BlockSpec((blk, _D), lambda i: (i, 0)),
            pl.BlockSpec((2, blk, _D), lambda i: (0, i, 0)),
            pl.BlockSpec((2, blk, _D), lambda i: (0, i, 0)),
            pl.BlockSpec((1, 1, blk), lambda i: (i, 0, 0)),
            pl.BlockSpec((_D, _HID), lambda i: (0, 0)),
            pl.BlockSpec((1, _HID), lambda i: (0, 0)),
            pl.BlockSpec((_HID, _NC), lambda i: (0, 0)),
            pl.BlockSpec((1, _NC), lambda i: (0, 0)),
        ],
        out_specs=pl.BlockSpec((_G, _NC), lambda i: (0, 0)),
        out_shape=jax.ShapeDtypeStruct((_G, _NC), jnp.float32),
        scratch_shapes=[
            pltpu.VMEM((_G, _D), jnp.float32),
            pltpu.VMEM((_G, _D), jnp.float32),
        ],
    )(h1, parts2, cnt4, batch3, w1, b1, w2, b2)


@jax.jit
def kernel(x, edge_index, edge_attr, batch, Wf1, bf1, Ws1, bs1,
           Wf2, bf2, Ws2, bs2, W1, b1, W2, b2):
    wd1 = jnp.concatenate([Wf1[:_D], Ws1[:_D]], axis=1)
    wsr1 = jnp.concatenate([Wf1[_D:2 * _D], Ws1[_D:2 * _D]], axis=1)
    we1 = jnp.concatenate([Wf1[2 * _D:], Ws1[2 * _D:]], axis=1)
    bb1 = jnp.concatenate([bf1, bs1]).reshape(1, 2 * _D)
    wd2 = jnp.concatenate([Wf2[:_D], Ws2[:_D]], axis=1)
    wsr2 = jnp.concatenate([Wf2[_D:2 * _D], Ws2[_D:2 * _D]], axis=1)
    we2 = jnp.concatenate([Wf2[2 * _D:], Ws2[2 * _D:]], axis=1)
    bb2 = jnp.concatenate([bf2, bs2]).reshape(1, 2 * _D)

    src = edge_index[0]
    dst = edge_index[1]
    zrows = jnp.zeros((_NPAD, _D), jnp.float32)
    batch3 = batch.reshape(_N // 1000, 1, 1000)

    td1, ts1 = _tc_tables(x, wd1, wsr1)
    ea1, ea2 = _tc_edge_terms(edge_attr, we1, bb1, we2, bb2)
    cparts = _sc_count_pass(dst, zrows)
    parts1 = _sc_edge_pass(td1, ts1, ea1, dst, src, zrows)
    h1, td2, ts2 = _tc_combine(x, parts1, cparts, wd2, wsr2)
    parts2 = _sc_edge_pass(td2, ts2, ea2, dst, src, zrows)
    return _tc_final(h1, parts2, cparts, batch3, W1, b1.reshape(1, _HID),
                     W2, b2.reshape(1, _NC))


# speedup vs baseline: 7.1171x; 5.6977x over previous
import functools
import jax
import jax.numpy as jnp
from jax import lax
from jax.experimental import pallas as pl
from jax.experimental.pallas import tpu as pltpu
from jax.experimental.pallas import tpu_sc as plsc

_N = 10000
_E = 320000
_D = 128
_DE = 16
_HID = 256
_NC = 10
_G = 64

_NPAD = 10240
_CROWS = _NPAD // 128
_NW = 32
_EPW = _E // _NW
_CH = 40
_ITERS = _EPW // _CH
_RPT = _NPAD // 16

_K0 = 0.99988785788851
_K1 = -0.496367575819865
_K2 = 0.304670274496034
_K3 = -0.15602615495847258
_K4 = 0.04106371768554667


def _pack_bf16_pair(f_part, s_part):
    uf = lax.bitcast_convert_type(f_part, jnp.uint32)
    us = lax.bitcast_convert_type(s_part, jnp.uint32)
    rf = (uf + jnp.uint32(0x7FFF) + ((uf >> 16) & jnp.uint32(1))) >> 16
    rs = (us + jnp.uint32(0x7FFF) + ((us >> 16) & jnp.uint32(1))) >> 16
    return lax.bitcast_convert_type((rs << 16) | rf, jnp.int32)


def _tables_body(x_ref, wd_ref, ws_ref, td_ref, ts_ref):
    xb = x_ref[...]
    td = jnp.dot(xb, wd_ref[...], preferred_element_type=jnp.float32)
    ts = jnp.dot(xb, ws_ref[...], preferred_element_type=jnp.float32)
    td_ref[...] = _pack_bf16_pair(td[:, :_D], td[:, _D:])
    ts_ref[...] = _pack_bf16_pair(ts[:, :_D], ts[:, _D:])


def _tc_tables(x, wd, ws):
    blk = 2000
    grid = _N // blk
    return pl.pallas_call(
        _tables_body,
        grid=(grid,),
        in_specs=[
            pl.BlockSpec((blk, _D), lambda i: (i, 0)),
            pl.BlockSpec((_D, 2 * _D), lambda i: (0, 0)),
            pl.BlockSpec((_D, 2 * _D), lambda i: (0, 0)),
        ],
        out_specs=[
            pl.BlockSpec((blk, _D), lambda i: (i, 0)),
            pl.BlockSpec((blk, _D), lambda i: (i, 0)),
        ],
        out_shape=[
            jax.ShapeDtypeStruct((_N, _D), jnp.int32),
            jax.ShapeDtypeStruct((_N, _D), jnp.int32),
        ],
    )(x, wd, ws)


def _ea1_body(ea_ref, we1_ref, bb1_ref, d3_ref, o1_ref, cnt_ref, cacc):
    i = pl.program_id(0)
    eb = ea_ref[...].astype(jnp.bfloat16)
    o1 = (jnp.dot(eb, we1_ref[...].astype(jnp.bfloat16),
                  preferred_element_type=jnp.float32) + bb1_ref[...])
    o1_ref[...] = _pack_bf16_pair(o1[:, :_D], o1[:, _D:])

    dstb = d3_ref[0, 0, :].reshape(-1, 1)
    rid = lax.broadcasted_iota(jnp.int32, (dstb.shape[0], _CROWS), 1)
    cid = lax.broadcasted_iota(jnp.int32, (dstb.shape[0], _D), 1)
    oh_r = ((dstb >> 7) == rid).astype(jnp.float32)
    oh_c = ((dstb & 127) == cid).astype(jnp.float32)
    dnum = (((0,), (0,)), ((), ()))
    blkcnt = lax.dot_general(oh_r, oh_c, dnum, preferred_element_type=jnp.float32)

    @pl.when(i == 0)
    def _():
        cacc[...] = jnp.zeros_like(cacc)

    cacc[...] += blkcnt

    @pl.when(i == pl.num_programs(0) - 1)
    def _():
        cnt_ref[...] = cacc[...]


def _ea2_body(ea_ref, we2_ref, bb2_ref, o2_ref):
    eb = ea_ref[...].astype(jnp.bfloat16)
    o2 = (jnp.dot(eb, we2_ref[...].astype(jnp.bfloat16),
                  preferred_element_type=jnp.float32) + bb2_ref[...])
    o2_ref[...] = _pack_bf16_pair(o2[:, :_D], o2[:, _D:])


def _tc_edge_terms1(edge_attr, we1, bb1, d3):
    blk = 4000
    grid = _E // blk
    return pl.pallas_call(
        _ea1_body,
        grid=(grid,),
        in_specs=[
            pl.BlockSpec((blk, _DE), lambda i: (i, 0)),
            pl.BlockSpec((_DE, 2 * _D), lambda i: (0, 0)),
            pl.BlockSpec((1, 2 * _D), lambda i: (0, 0)),
            pl.BlockSpec((1, 1, blk), lambda i: (i, 0, 0)),
        ],
        out_specs=[
            pl.BlockSpec((blk, _D), lambda i: (i, 0)),
            pl.BlockSpec((_CROWS, _D), lambda i: (0, 0)),
        ],
        out_shape=[
            jax.ShapeDtypeStruct((_E, _D), jnp.int32),
            jax.ShapeDtypeStruct((_CROWS, _D), jnp.float32),
        ],
        scratch_shapes=[pltpu.VMEM((_CROWS, _D), jnp.float32)],
    )(edge_attr, we1, bb1, d3)


def _tc_edge_terms2(edge_attr, we2, bb2):
    blk = 4000
    grid = _E // blk
    return pl.pallas_call(
        _ea2_body,
        grid=(grid,),
        in_specs=[
            pl.BlockSpec((blk, _DE), lambda i: (i, 0)),
            pl.BlockSpec((_DE, 2 * _D), lambda i: (0, 0)),
            pl.BlockSpec((1, 2 * _D), lambda i: (0, 0)),
        ],
        out_specs=pl.BlockSpec((blk, _D), lambda i: (i, 0)),
        out_shape=jax.ShapeDtypeStruct((_E, _D), jnp.int32),
    )(edge_attr, we2, bb2)


def _msg_quad(aa, bb):
    n = len(aa)
    tt = [jnp.exp(-a) for a in aa]
    uu = [jnp.exp(-jnp.abs(b)) for b in bb]
    ff = [1.0 / (1.0 + t) for t in tt]
    ss = [_K0 + _K1 * u for u in uu]
    rr = [_K2 + _K3 * u for u in uu]
    u2 = [u * u for u in uu]
    ww = [rr[i] + u2[i] * _K4 for i in range(n)]
    pp = [ss[i] + u2[i] * ww[i] for i in range(n)]
    sp = [jnp.maximum(bb[i], 0.0) + uu[i] * pp[i] for i in range(n)]
    return [ff[i] * sp[i] for i in range(n)]


def _sc_edge_body(td_hbm, ts_hbm, ea_hbm, dst_hbm, src_hbm, zero_hbm,
                  out_hbm, dstv, srcv, dsts, gd, gs, ea, mbuf, acc,
                  semi, semg, sems):
    c = lax.axis_index("c")
    s = lax.axis_index("s")
    wid = s * 2 + c
    base = wid * _EPW
    last = _ITERS - 1

    pltpu.sync_copy(zero_hbm.at[pl.ds(s * _RPT, _RPT)], acc.at[pl.ds(s * _RPT, _RPT)])
    plsc.subcore_barrier()

    def idx_start(i, p):
        off = base + i * _CH
        pltpu.async_copy(dst_hbm.at[pl.ds(off, _CH)], dstv.at[p], semi.at[p])
        pltpu.async_copy(src_hbm.at[pl.ds(off, _CH)], srcv.at[p], semi.at[p])

    def idx_wait(p):
        pltpu.make_async_copy(dst_hbm.at[pl.ds(0, _CH)], dstv.at[p], semi.at[p]).wait()
        pltpu.make_async_copy(src_hbm.at[pl.ds(0, _CH)], srcv.at[p], semi.at[p]).wait()

    def gather_start(i, p):
        off = base + i * _CH
        pltpu.async_copy(td_hbm.at[dstv.at[p]], gd.at[p], semg.at[p])
        pltpu.async_copy(ts_hbm.at[srcv.at[p]], gs.at[p], semg.at[p])
        pltpu.async_copy(ea_hbm.at[pl.ds(off, _CH)], ea.at[p], semg.at[p])

    def gather_wait(p):
        pltpu.make_async_copy(td_hbm.at[dstv.at[p]], gd.at[p], semg.at[p]).wait()
        pltpu.make_async_copy(ts_hbm.at[srcv.at[p]], gs.at[p], semg.at[p]).wait()
        pltpu.make_async_copy(ea_hbm.at[pl.ds(0, _CH)], ea.at[p], semg.at[p]).wait()

    def scatter_start(p):
        pltpu.async_copy(mbuf.at[p], acc.at[dsts.at[p]], sems.at[p], add=True)

    def scatter_wait(p):
        pltpu.make_async_copy(mbuf.at[p], acc.at[dsts.at[p]], sems.at[p]).wait()

    def grp(e, p, k):
        sl = pl.ds(16 * k, 16)
        wd_ = gd[p, e, sl]
        ws_ = gs[p, e, sl]
        we_ = ea[p, e, sl]
        bc = lambda v: lax.bitcast_convert_type(v, jnp.float32)
        a = bc(wd_ << 16) + bc(ws_ << 16) + bc(we_ << 16)
        b = bc(wd_) + bc(ws_) + bc(we_)
        return a, b

    def compute(p):
        def edge2(jj, _):
            for e in (2 * jj, 2 * jj + 1):
                for k0 in (0, 4):
                    ab = [grp(e, p, k0 + k) for k in range(4)]
                    mm = _msg_quad([x[0] for x in ab], [x[1] for x in ab])
                    for k in range(4):
                        mbuf[p, e, pl.ds(16 * (k0 + k), 16)] = mm[k]
            return _

        lax.fori_loop(0, _CH // 2, edge2, None)

    idx_start(0, 0)
    idx_wait(0)
    gather_start(0, 0)
    idx_start(1, 1)

    def pair(k, _):
        for p in (0, 1):
            i = 2 * k + p

            @pl.when((i >= 2) & (i <= last))
            def _():
                scatter_wait(p)

            @pl.when(i <= last)
            def _():
                gather_wait(p)
                dsts[p, pl.ds(0, _CH)] = dstv[p, pl.ds(0, _CH)]

            @pl.when(i + 2 <= last)
            def _():
                idx_start(i + 2, p)

            @pl.when(i + 1 <= last)
            def _():
                idx_wait(1 - p)
                gather_start(i + 1, 1 - p)

            @pl.when(i <= last)
            def _():
                compute(p)
                scatter_start(p)
        return _

    lax.fori_loop(0, (_ITERS + 1) // 2, pair, None)
    scatter_wait(last % 2)
    scatter_wait(1 - last % 2)
    plsc.subcore_barrier()
    pltpu.sync_copy(acc.at[pl.ds(s * _RPT, _RPT)],
                    out_hbm.at[c, pl.ds(s * _RPT, _RPT)])


def _sc_edge_pass(td, ts, eat, dst, src, zrows):
    mesh = plsc.VectorSubcoreMesh(core_axis_name="c", subcore_axis_name="s")
    kfn = pl.kernel(
        _sc_edge_body,
        mesh=mesh,
        out_type=jax.ShapeDtypeStruct((2, _NPAD, _D), jnp.float32),
        scratch_types=[
            pltpu.VMEM((2, _CH), jnp.int32),
            pltpu.VMEM((2, _CH), jnp.int32),
            pltpu.VMEM((2, _CH), jnp.int32),
            pltpu.VMEM((2, _CH, _D), jnp.int32),
            pltpu.VMEM((2, _CH, _D), jnp.int32),
            pltpu.VMEM((2, _CH, _D), jnp.int32),
            pltpu.VMEM((2, _CH, _D), jnp.float32),
            pltpu.VMEM_SHARED((_NPAD, _D), jnp.float32),
            pltpu.SemaphoreType.DMA((2,)),
            pltpu.SemaphoreType.DMA((2,)),
            pltpu.SemaphoreType.DMA((2,)),
        ],
    )
    return kfn(td, ts, eat, dst, src, zrows)


def _combine_body(x_ref, p_ref, c_ref, wd_ref, ws_ref, h_ref, td_ref, ts_ref):
    p = p_ref[0] + p_ref[1]
    cnt = c_ref[0, 0, :].reshape(-1, 1)
    agg = p / jnp.maximum(cnt, 1.0)
    h = x_ref[...] + agg
    h_ref[...] = h
    td = jnp.dot(h, wd_ref[...], preferred_element_type=jnp.float32)
    ts = jnp.dot(h, ws_ref[...], preferred_element_type=jnp.float32)
    td_ref[...] = _pack_bf16_pair(td[:, :_D], td[:, _D:])
    ts_ref[...] = _pack_bf16_pair(ts[:, :_D], ts[:, _D:])


def _tc_combine(x, parts, cnt4, wd2, ws2):
    blk = 2000
    grid = _N // blk
    return pl.pallas_call(
        _combine_body,
        grid=(grid,),
        in_specs=[
            pl.BlockSpec((blk, _D), lambda i: (i, 0)),
            pl.BlockSpec((2, blk, _D), lambda i: (0, i, 0)),
            pl.BlockSpec((1, 1, blk), lambda i: (i, 0, 0)),
            pl.BlockSpec((_D, 2 * _D), lambda i: (0, 0)),
            pl.BlockSpec((_D, 2 * _D), lambda i: (0, 0)),
        ],
        out_specs=[
            pl.BlockSpec((blk, _D), lambda i: (i, 0)),
            pl.BlockSpec((blk, _D), lambda i: (i, 0)),
            pl.BlockSpec((blk, _D), lambda i: (i, 0)),
        ],
        out_shape=[
            jax.ShapeDtypeStruct((_N, _D), jnp.float32),
            jax.ShapeDtypeStruct((_N, _D), jnp.int32),
            jax.ShapeDtypeStruct((_N, _D), jnp.int32),
        ],
    )(x, parts, cnt4, wd2, ws2)


def _final_body(h1_ref, q_ref, cq_ref, b_ref, w1_ref, b1_ref, w2_ref, b2_ref,
                o_ref, s_acc, c_acc):
    i = pl.program_id(0)
    q = q_ref[0] + q_ref[1]
    cnt = cq_ref[0, 0, :].reshape(-1, 1)
    h2 = h1_ref[...] + q / jnp.maximum(cnt, 1.0)

    batch = b_ref[0, 0, :].reshape(-1, 1)
    gids = lax.broadcasted_iota(jnp.int32, (batch.shape[0], _G), 1)
    oh = (batch == gids).astype(jnp.float32)
    dnum = (((0,), (0,)), ((), ()))
    sblk = lax.dot_general(oh, h2, dnum, preferred_element_type=jnp.float32)
    cblk = lax.dot_general(oh, jnp.ones_like(h2), dnum,
                           preferred_element_type=jnp.float32)

    @pl.when(i == 0)
    def _():
        s_acc[...] = jnp.zeros_like(s_acc)
        c_acc[...] = jnp.zeros_like(c_acc)

    s_acc[...] += sblk
    c_acc[...] += cblk

    @pl.when(i == pl.num_programs(0) - 1)
    def _():
        g = s_acc[...] / jnp.maximum(c_acc[...], 1.0)
        a1 = jnp.maximum(
            jnp.dot(g, w1_ref[...], preferred_element_type=jnp.float32)
            + b1_ref[...], 0.0)
        o = (jnp.dot(a1, w2_ref[...], preferred_element_type=jnp.float32)
             + b2_ref[...])
        mx = jnp.max(o, axis=1, keepdims=True)
        lse = jnp.log(jnp.sum(jnp.exp(o - mx), axis=1, keepdims=True)) + mx
        o_ref[...] = o - lse


def _tc_final(h1, parts2, cnt4, batch3, w1, b1, w2, b2):
    blk = 1000
    grid = _N // blk
    return pl.pallas_call(
        _final_body,
        grid=(grid,),
        in_specs=[
            pl.BlockSpec((blk, _D), lambda i: (i, 0)),
            pl.BlockSpec((2, blk, _D), lambda i: (0, i, 0)),
            pl.BlockSpec((1, 1, blk), lambda i: (i, 0, 0)),
            pl.BlockSpec((1, 1, blk), lambda i: (i, 0, 0)),
            pl.BlockSpec((_D, _HID), lambda i: (0, 0)),
            pl.BlockSpec((1, _HID), lambda i: (0, 0)),
            pl.BlockSpec((_HID, _NC), lambda i: (0, 0)),
            pl.BlockSpec((1, _NC), lambda i: (0, 0)),
        ],
        out_specs=pl.BlockSpec((_G, _NC), lambda i: (0, 0)),
        out_shape=jax.ShapeDtypeStruct((_G, _NC), jnp.float32),
        scratch_shapes=[
            pltpu.VMEM((_G, _D), jnp.float32),
            pltpu.VMEM((_G, _D), jnp.float32),
        ],
    )(h1, parts2, cnt4, batch3, w1, b1, w2, b2)


@jax.jit
def kernel(x, edge_index, edge_attr, batch, Wf1, bf1, Ws1, bs1,
           Wf2, bf2, Ws2, bs2, W1, b1, W2, b2):
    wd1 = jnp.concatenate([Wf1[:_D], Ws1[:_D]], axis=1)
    wsr1 = jnp.concatenate([Wf1[_D:2 * _D], Ws1[_D:2 * _D]], axis=1)
    we1 = jnp.concatenate([Wf1[2 * _D:], Ws1[2 * _D:]], axis=1)
    bb1 = jnp.concatenate([bf1, bs1]).reshape(1, 2 * _D)
    wd2 = jnp.concatenate([Wf2[:_D], Ws2[:_D]], axis=1)
    wsr2 = jnp.concatenate([Wf2[_D:2 * _D], Ws2[_D:2 * _D]], axis=1)
    we2 = jnp.concatenate([Wf2[2 * _D:], Ws2[2 * _D:]], axis=1)
    bb2 = jnp.concatenate([bf2, bs2]).reshape(1, 2 * _D)

    src = edge_index[0]
    dst = edge_index[1]
    zrows = jnp.zeros((_NPAD, _D), jnp.float32)
    batch3 = batch.reshape(_N // 1000, 1, 1000)

    d3 = dst.reshape(_E // 4000, 1, 4000)
    td1, ts1 = _tc_tables(x, wd1, wsr1)
    ea1, cnt2d = _tc_edge_terms1(edge_attr, we1, bb1, d3)
    cnt1d = cnt2d.reshape(_NPAD)[:_N]
    cnt3 = cnt1d.reshape(_N // 1000, 1, 1000)
    cnt3w = cnt1d.reshape(_N // 2000, 1, 2000)
    parts1 = _sc_edge_pass(td1, ts1, ea1, dst, src, zrows)
    ea2 = _tc_edge_terms2(edge_attr, we2, bb2)
    h1, td2, ts2 = _tc_combine(x, parts1, cnt3w, wd2, wsr2)
    parts2 = _sc_edge_pass(td2, ts2, ea2, dst, src, zrows)
    return _tc_final(h1, parts2, cnt3, batch3, W1, b1.reshape(1, _HID),
                     W2, b2.reshape(1, _NC))
